# baseline (speedup 1.0000x reference)
import numpy as np

import jax
import jax.numpy as jnp
from jax.experimental import pallas as pl
from jax.experimental.pallas import tpu as pltpu

_ANT_IDX = (3, 7, 12, 20)
_CONS_IDX = (45, 60, 77)
_SIGNS_A = (1.0, -1.0, 1.0, -1.0)
_SIGNS_C = (1.0, -1.0, 1.0)
_ALL_IDX = _ANT_IDX + _CONS_IDX
_MIN_W, _MAX_W = 0.0, 500.0

_N_PRED = 128
_BM = 2048


def _consts():
    sv = np.zeros((1, _N_PRED), np.float32)
    dv = np.zeros((1, _N_PRED), np.float32)
    msk = np.zeros((1, _N_PRED), np.float32)
    for c, s in zip(_ANT_IDX, _SIGNS_A):
        sv[0, c] = -s
        dv[0, c] = -s
        msk[0, c] = 1.0
    for c, s in zip(_CONS_IDX, _SIGNS_C):
        sv[0, c] = s
        dv[0, c] = s
        msk[0, c] = 1.0
    ma = np.zeros((1, _N_PRED), np.float32)
    mc = np.zeros((1, _N_PRED), np.float32)
    for c in _ANT_IDX:
        ma[0, c] = 1.0
    for c in _CONS_IDX:
        mc[0, c] = 1.0
    pm = np.zeros((_N_PRED, 8), np.float32)
    for k, c in enumerate(_ALL_IDX):
        pm[c, k] = 1.0
    return jnp.asarray(sv), jnp.asarray(dv), jnp.asarray(msk), \
        jnp.asarray(ma), jnp.asarray(mc), jnp.asarray(pm)


def _body(w_ref, x_ref, sv_ref, dv_ref, msk_ref, ma_ref, mc_ref, pm_ref,
          out_ref, delta_ref):
    w = jnp.clip(w_ref[0, 0], _MIN_W, _MAX_W)
    x = x_ref[...]
    msk = msk_ref[...]
    ma = ma_ref[...]
    mc = mc_ref[...]
    e = jnp.exp(x * sv_ref[...]) * msk
    sa = jnp.sum(e * ma, axis=1, keepdims=True)
    sc = jnp.sum(e * mc, axis=1, keepdims=True)
    denom = sa * ma + sc * mc + (1.0 - msk)
    out = (e / denom) * (dv_ref[...] * w)
    out_ref[...] = out
    d8 = jax.lax.dot_general(
        out, pm_ref[...], (((1,), (0,)), ((), ())),
        precision=jax.lax.Precision.HIGHEST,
        preferred_element_type=jnp.float32)
    delta_ref[...] = d8[:, :len(_ALL_IDX)]


def kernel(ground_atoms, clause_weight):
    batch, n_pred = ground_atoms.shape
    w2d = clause_weight.reshape(1, 1)
    sv, dv, msk, ma, mc, pm = _consts()
    grid = (batch // _BM,)
    pmspec = pl.BlockSpec((_N_PRED, 8), lambda i: (0, 0))
    row = pl.BlockSpec((1, _N_PRED), lambda i: (0, 0))
    out, delta = pl.pallas_call(
        _body,
        grid=grid,
        in_specs=[
            pl.BlockSpec(memory_space=pltpu.SMEM),
            pl.BlockSpec((_BM, n_pred), lambda i: (i, 0)),
            row, row, row, row, row, pmspec,
        ],
        out_specs=[
            pl.BlockSpec((_BM, n_pred), lambda i: (i, 0)),
            pl.BlockSpec((_BM, len(_ALL_IDX)), lambda i: (i, 0)),
        ],
        out_shape=[
            jax.ShapeDtypeStruct((batch, n_pred), jnp.float32),
            jax.ShapeDtypeStruct((batch, len(_ALL_IDX)), jnp.float32),
        ],
    )(w2d, ground_atoms, sv, dv, msk, ma, mc, pm)
    return (out, delta)
